# baseline (speedup 1.0000x reference)
import jax
import jax.numpy as jnp
from jax import lax
from jax.experimental import pallas as pl
from jax.experimental.pallas import tpu as pltpu

_EPS = 1e-5
_B = 16
_H1 = 32
_GROWTH = 32

_TAPS = [(dh, dw) for dh in (-1, 0, 1) for dw in (-1, 0, 1)]


def _shift_lanes(a, off, ntot):
    s = off % ntot
    if s == 0:
        return a
    return jnp.concatenate([a[:, s:], a[:, :s]], axis=1)


def _tap_mask(hh, ww, dh, dw, H, W):
    valid = None
    def _and(v, c):
        return c if v is None else (v & c)
    if dh < 0:
        valid = _and(valid, hh + dh >= 0)
    elif dh > 0:
        valid = _and(valid, hh + dh < H)
    if dw < 0:
        valid = _and(valid, ww + dw >= 0)
    elif dw > 0:
        valid = _and(valid, ww + dw < W)
    return valid


def _spatial_iota(H, W, ntot):
    lane = lax.broadcasted_iota(jnp.int32, (1, ntot), 1)
    pix = lane % (H * W)
    return pix // W, pix % W


def _conv3x3(a2, wA, growth, H, W, A_ref):
    ntot = a2.shape[1]
    hh, ww = _spatial_iota(H, W, ntot)
    zero = jnp.zeros((), a2.dtype)
    A_ref[0:128, 0:ntot] = jnp.where(ww > 0, _shift_lanes(a2, -1, ntot), zero)
    A_ref[256:384, 0:ntot] = jnp.where(ww < W - 1, _shift_lanes(a2, 1, ntot),
                                       zero)
    inner = jnp.dot(wA, A_ref[:, 0:ntot], preferred_element_type=jnp.float32)
    acc = inner[growth:2 * growth]
    up = _shift_lanes(inner[0:growth], -W, ntot)
    acc = acc + jnp.where(hh > 0, up, 0.0)
    dn = _shift_lanes(inner[2 * growth:3 * growth], W, ntot)
    return acc + jnp.where(hh < H - 1, dn, 0.0)


def _stack9(xa, H, W):
    ntot = xa.shape[1]
    hh, ww = _spatial_iota(H, W, ntot)
    zero = jnp.zeros((), xa.dtype)
    b_m1 = jnp.where(ww > 0, _shift_lanes(xa, -1, ntot), zero)
    b_p1 = jnp.where(ww < W - 1, _shift_lanes(xa, 1, ntot), zero)
    a9 = jnp.concatenate([b_m1, xa, b_p1], axis=0)
    up = jnp.where(hh > 0, _shift_lanes(a9, -W, ntot), zero)
    dn = jnp.where(hh < H - 1, _shift_lanes(a9, W, ntot), zero)
    return jnp.concatenate([up, a9, dn], axis=0)


def _dense_block(slab, s1, b1, w1, wA, H, W, A_ref):
    zb = jnp.zeros((), jnp.bfloat16)
    ntot = slab.shape[1]
    h = jnp.maximum(slab * s1 + b1, zb)
    ones = jnp.ones((1, ntot), jnp.bfloat16)
    A_ref[128:256, 0:ntot] = jnp.maximum(
        jnp.dot(w1, jnp.concatenate([h, ones], axis=0),
                preferred_element_type=jnp.float32).astype(jnp.bfloat16), zb)
    a2 = A_ref[128:256, 0:ntot]
    acc = _conv3x3(a2, wA, _GROWTH, H, W, A_ref)
    return jnp.concatenate([slab, acc.astype(jnp.bfloat16)], axis=0)


def _transition(slab, s, b, wt, poolT, HW):
    h = jnp.maximum(slab * s + b, jnp.zeros((), jnp.bfloat16))
    y1 = jnp.dot(wt, h, preferred_element_type=jnp.float32).astype(jnp.bfloat16)
    pieces = [
        jnp.dot(y1[:, i * HW:(i + 1) * HW], poolT,
                preferred_element_type=jnp.float32).astype(jnp.bfloat16)
        for i in range(_B)
    ]
    return jnp.concatenate(pieces, axis=1)


def _net_kernel(x_ref, w27_ref,
                d1s1_ref, d1b1_ref, d1w1_ref, d1wA_ref,
                t1s_ref, t1b_ref, t1w_ref, p1_ref,
                d2s1_ref, d2b1_ref, d2w1_ref, d2wA_ref,
                t2s_ref, t2b_ref, t2w_ref, p2_ref,
                d3s1_ref, d3b1_ref, d3w1_ref, d3wA_ref,
                hs_ref, hb_ref, fcw_ref, fcb_ref, mean_ref,
                o_ref, A_ref):
    H = W = _H1
    xa = x_ref[0]
    stacked = _stack9(xa, H, W)
    stem = jnp.dot(w27_ref[...], stacked,
                   preferred_element_type=jnp.float32).astype(jnp.bfloat16)

    slab = _dense_block(stem, d1s1_ref[...], d1b1_ref[...], d1w1_ref[...],
                        d1wA_ref[...], H, W, A_ref)
    slab = _transition(slab, t1s_ref[...], t1b_ref[...], t1w_ref[...],
                       p1_ref[...], H * W)
    H = W = _H1 // 2
    slab = _dense_block(slab, d2s1_ref[...], d2b1_ref[...], d2w1_ref[...],
                        d2wA_ref[...], H, W, A_ref)
    slab = _transition(slab, t2s_ref[...], t2b_ref[...], t2w_ref[...],
                       p2_ref[...], H * W)
    H = W = _H1 // 4
    slab = _dense_block(slab, d3s1_ref[...], d3b1_ref[...], d3w1_ref[...],
                        d3wA_ref[...], H, W, A_ref)

    hf = jnp.maximum(slab.astype(jnp.float32) * hs_ref[...] + hb_ref[...], 0.0)
    pooled = jnp.dot(hf, mean_ref[...], preferred_element_type=jnp.float32)
    logits = jnp.dot(fcw_ref[...], pooled,
                     preferred_element_type=jnp.float32) + fcb_ref[...]
    m = jnp.max(logits, axis=0, keepdims=True)
    z = logits - m
    lse = jnp.log(jnp.sum(jnp.exp(z), axis=0, keepdims=True))
    o_ref[0] = z - lse


def _bn_scale_shift(gamma, beta, mean, var):
    scale = gamma / jnp.sqrt(var + _EPS)
    shift = beta - mean * scale
    return scale.reshape(-1, 1), shift.reshape(-1, 1)


def _dh_stacked_taps(w):
    g, cin = w.shape[0], w.shape[1]
    return jnp.transpose(w, (2, 0, 3, 1)).reshape(3 * g, 3 * cin)


def _pool_matrix(H, W):
    HW, HWp = H * W, (H // 2) * (W // 2)
    r = jnp.arange(HW)
    rp = (r // W // 2) * (W // 2) + (r % W) // 2
    return (rp[:, None] == jnp.arange(HWp)[None, :]).astype(jnp.float32) * 0.25


def kernel(x, conv1_w, dense1_0_bn1_gamma, dense1_0_bn1_beta, dense1_0_bn1_mean, dense1_0_bn1_var, dense1_0_bn2_gamma, dense1_0_bn2_beta, dense1_0_bn2_mean, dense1_0_bn2_var, dense1_0_conv1_w, dense1_0_conv2_w, dense2_0_bn1_gamma, dense2_0_bn1_beta, dense2_0_bn1_mean, dense2_0_bn1_var, dense2_0_bn2_gamma, dense2_0_bn2_beta, dense2_0_bn2_mean, dense2_0_bn2_var, dense2_0_conv1_w, dense2_0_conv2_w, dense3_0_bn1_gamma, dense3_0_bn1_beta, dense3_0_bn1_mean, dense3_0_bn1_var, dense3_0_bn2_gamma, dense3_0_bn2_beta, dense3_0_bn2_mean, dense3_0_bn2_var, dense3_0_conv1_w, dense3_0_conv2_w, trans1_bn_gamma, trans1_bn_beta, trans1_bn_mean, trans1_bn_var, trans1_conv_w, trans2_bn_gamma, trans2_bn_beta, trans2_bn_mean, trans2_bn_var, trans2_conv_w, bn1_gamma, bn1_beta, bn1_mean, bn1_var, fc_w, fc_b):
    N = x.shape[0]
    G = N // _B
    HW1 = _H1 * _H1
    n1 = _B * HW1

    bf = jnp.bfloat16
    xg = (x.reshape(N, 3, HW1).astype(bf)
           .reshape(G, _B, 3, HW1).transpose(0, 2, 1, 3).reshape(G, 3, n1))

    w27 = jnp.transpose(conv1_w, (0, 2, 3, 1)).reshape(conv1_w.shape[0], 27).astype(bf)

    def _dense_prep(bn1g, bn1b, bn1m, bn1v, bn2g, bn2b, bn2m, bn2v, w1, w2):
        s1, b1 = _bn_scale_shift(bn1g, bn1b, bn1m, bn1v)
        s1, b1 = s1.astype(bf), b1.astype(bf)
        s2, b2 = _bn_scale_shift(bn2g, bn2b, bn2m, bn2v)
        w1f = jnp.concatenate([s2 * w1[:, :, 0, 0], b2], axis=1).astype(bf)
        wA = _dh_stacked_taps(w2).astype(bf)
        return s1, b1, w1f, wA

    d1s1, d1b1, d1w1, d1wA = _dense_prep(
        dense1_0_bn1_gamma, dense1_0_bn1_beta, dense1_0_bn1_mean,
        dense1_0_bn1_var, dense1_0_bn2_gamma, dense1_0_bn2_beta,
        dense1_0_bn2_mean, dense1_0_bn2_var, dense1_0_conv1_w, dense1_0_conv2_w)
    d2s1, d2b1, d2w1, d2wA = _dense_prep(
        dense2_0_bn1_gamma, dense2_0_bn1_beta, dense2_0_bn1_mean,
        dense2_0_bn1_var, dense2_0_bn2_gamma, dense2_0_bn2_beta,
        dense2_0_bn2_mean, dense2_0_bn2_var, dense2_0_conv1_w, dense2_0_conv2_w)
    d3s1, d3b1, d3w1, d3wA = _dense_prep(
        dense3_0_bn1_gamma, dense3_0_bn1_beta, dense3_0_bn1_mean,
        dense3_0_bn1_var, dense3_0_bn2_gamma, dense3_0_bn2_beta,
        dense3_0_bn2_mean, dense3_0_bn2_var, dense3_0_conv1_w, dense3_0_conv2_w)

    t1s, t1b = _bn_scale_shift(trans1_bn_gamma, trans1_bn_beta,
                               trans1_bn_mean, trans1_bn_var)
    t1s, t1b = t1s.astype(bf), t1b.astype(bf)
    t1w = trans1_conv_w[:, :, 0, 0].astype(bf)
    p1 = _pool_matrix(_H1, _H1).astype(bf)
    t2s, t2b = _bn_scale_shift(trans2_bn_gamma, trans2_bn_beta,
                               trans2_bn_mean, trans2_bn_var)
    t2s, t2b = t2s.astype(bf), t2b.astype(bf)
    t2w = trans2_conv_w[:, :, 0, 0].astype(bf)
    p2 = _pool_matrix(_H1 // 2, _H1 // 2).astype(bf)

    hs, hb = _bn_scale_shift(bn1_gamma, bn1_beta, bn1_mean, bn1_var)
    nC, Ch = fc_w.shape
    HW3 = (_H1 // 4) * (_H1 // 4)
    mean_m = (jnp.repeat(jnp.eye(_B, dtype=jnp.float32), HW3, axis=0) / HW3)

    full = lambda shp: pl.BlockSpec(shp, lambda g: (0,) * len(shp))
    out = pl.pallas_call(
        _net_kernel,
        out_shape=jax.ShapeDtypeStruct((G, nC, _B), jnp.float32),
        grid=(G,),
        in_specs=[
            pl.BlockSpec((1, 3, n1), lambda g: (g, 0, 0)),
            full(w27.shape),
            full(d1s1.shape), full(d1b1.shape), full(d1w1.shape), full(d1wA.shape),
            full(t1s.shape), full(t1b.shape), full(t1w.shape), full(p1.shape),
            full(d2s1.shape), full(d2b1.shape), full(d2w1.shape), full(d2wA.shape),
            full(t2s.shape), full(t2b.shape), full(t2w.shape), full(p2.shape),
            full(d3s1.shape), full(d3b1.shape), full(d3w1.shape), full(d3wA.shape),
            full(hs.shape), full(hb.shape), full(fc_w.shape),
            full((nC, 1)), full(mean_m.shape),
        ],
        out_specs=pl.BlockSpec((1, nC, _B), lambda g: (g, 0, 0)),
        scratch_shapes=[pltpu.VMEM((384, n1), jnp.bfloat16)],
        compiler_params=pltpu.CompilerParams(dimension_semantics=("parallel",)),
    )(xg, w27,
      d1s1, d1b1, d1w1, d1wA,
      t1s, t1b, t1w, p1,
      d2s1, d2b1, d2w1, d2wA,
      t2s, t2b, t2w, p2,
      d3s1, d3b1, d3w1, d3wA,
      hs, hb, fc_w, fc_b.reshape(nC, 1), mean_m)

    return out.transpose(0, 2, 1).reshape(N, nC)


# speedup vs baseline: 1.0730x; 1.0730x over previous
import jax
import jax.numpy as jnp
from jax import lax
from jax.experimental import pallas as pl
from jax.experimental.pallas import tpu as pltpu

_EPS = 1e-5
_B = 32
_H1 = 32
_GROWTH = 32

_TAPS = [(dh, dw) for dh in (-1, 0, 1) for dw in (-1, 0, 1)]


def _shift_lanes(a, off, ntot):
    s = off % ntot
    if s == 0:
        return a
    return jnp.concatenate([a[:, s:], a[:, :s]], axis=1)


def _tap_mask(hh, ww, dh, dw, H, W):
    valid = None
    def _and(v, c):
        return c if v is None else (v & c)
    if dh < 0:
        valid = _and(valid, hh + dh >= 0)
    elif dh > 0:
        valid = _and(valid, hh + dh < H)
    if dw < 0:
        valid = _and(valid, ww + dw >= 0)
    elif dw > 0:
        valid = _and(valid, ww + dw < W)
    return valid


def _spatial_iota(H, W, ntot):
    lane = lax.broadcasted_iota(jnp.int32, (1, ntot), 1)
    pix = lane % (H * W)
    return pix // W, pix % W


def _conv3x3(a2, wA, growth, H, W, A_ref):
    ntot = a2.shape[1]
    hh, ww = _spatial_iota(H, W, ntot)
    zero = jnp.zeros((), a2.dtype)
    A_ref[0:128, 0:ntot] = jnp.where(ww > 0, _shift_lanes(a2, -1, ntot), zero)
    A_ref[256:384, 0:ntot] = jnp.where(ww < W - 1, _shift_lanes(a2, 1, ntot),
                                       zero)
    inner = jnp.dot(wA, A_ref[:, 0:ntot], preferred_element_type=jnp.float32)
    acc = inner[growth:2 * growth]
    up = _shift_lanes(inner[0:growth], -W, ntot)
    acc = acc + jnp.where(hh > 0, up, 0.0)
    dn = _shift_lanes(inner[2 * growth:3 * growth], W, ntot)
    return acc + jnp.where(hh < H - 1, dn, 0.0)


def _stack9(xa, H, W):
    ntot = xa.shape[1]
    hh, ww = _spatial_iota(H, W, ntot)
    zero = jnp.zeros((), xa.dtype)
    b_m1 = jnp.where(ww > 0, _shift_lanes(xa, -1, ntot), zero)
    b_p1 = jnp.where(ww < W - 1, _shift_lanes(xa, 1, ntot), zero)
    a9 = jnp.concatenate([b_m1, xa, b_p1], axis=0)
    up = jnp.where(hh > 0, _shift_lanes(a9, -W, ntot), zero)
    dn = jnp.where(hh < H - 1, _shift_lanes(a9, W, ntot), zero)
    return jnp.concatenate([up, a9, dn], axis=0)


def _dense_block(slab, s1, b1, w1, wA, H, W, A_ref):
    zb = jnp.zeros((), jnp.bfloat16)
    ntot = slab.shape[1]
    h = jnp.maximum(slab * s1 + b1, zb)
    ones = jnp.ones((1, ntot), jnp.bfloat16)
    A_ref[128:256, 0:ntot] = jnp.maximum(
        jnp.dot(w1, jnp.concatenate([h, ones], axis=0),
                preferred_element_type=jnp.float32).astype(jnp.bfloat16), zb)
    a2 = A_ref[128:256, 0:ntot]
    acc = _conv3x3(a2, wA, _GROWTH, H, W, A_ref)
    return jnp.concatenate([slab, acc.astype(jnp.bfloat16)], axis=0)


def _transition(slab, s, b, wt, poolT, HW):
    h = jnp.maximum(slab * s + b, jnp.zeros((), jnp.bfloat16))
    y1 = jnp.dot(wt, h, preferred_element_type=jnp.float32).astype(jnp.bfloat16)
    pieces = [
        jnp.dot(y1[:, i * HW:(i + 1) * HW], poolT,
                preferred_element_type=jnp.float32).astype(jnp.bfloat16)
        for i in range(_B)
    ]
    return jnp.concatenate(pieces, axis=1)


def _net_kernel(x_ref, w27_ref,
                d1s1_ref, d1b1_ref, d1w1_ref, d1wA_ref,
                t1s_ref, t1b_ref, t1w_ref, p1_ref,
                d2s1_ref, d2b1_ref, d2w1_ref, d2wA_ref,
                t2s_ref, t2b_ref, t2w_ref, p2_ref,
                d3s1_ref, d3b1_ref, d3w1_ref, d3wA_ref,
                hs_ref, hb_ref, fcw_ref, fcb_ref, mean_ref,
                o_ref, A_ref):
    H = W = _H1
    xa = x_ref[0]
    stacked = _stack9(xa, H, W)
    stem = jnp.dot(w27_ref[...], stacked,
                   preferred_element_type=jnp.float32).astype(jnp.bfloat16)

    slab = _dense_block(stem, d1s1_ref[...], d1b1_ref[...], d1w1_ref[...],
                        d1wA_ref[...], H, W, A_ref)
    slab = _transition(slab, t1s_ref[...], t1b_ref[...], t1w_ref[...],
                       p1_ref[...], H * W)
    H = W = _H1 // 2
    slab = _dense_block(slab, d2s1_ref[...], d2b1_ref[...], d2w1_ref[...],
                        d2wA_ref[...], H, W, A_ref)
    slab = _transition(slab, t2s_ref[...], t2b_ref[...], t2w_ref[...],
                       p2_ref[...], H * W)
    H = W = _H1 // 4
    slab = _dense_block(slab, d3s1_ref[...], d3b1_ref[...], d3w1_ref[...],
                        d3wA_ref[...], H, W, A_ref)

    hf = jnp.maximum(slab.astype(jnp.float32) * hs_ref[...] + hb_ref[...], 0.0)
    pooled = jnp.dot(hf, mean_ref[...], preferred_element_type=jnp.float32)
    logits = jnp.dot(fcw_ref[...], pooled,
                     preferred_element_type=jnp.float32) + fcb_ref[...]
    m = jnp.max(logits, axis=0, keepdims=True)
    z = logits - m
    lse = jnp.log(jnp.sum(jnp.exp(z), axis=0, keepdims=True))
    o_ref[0] = z - lse


def _bn_scale_shift(gamma, beta, mean, var):
    scale = gamma / jnp.sqrt(var + _EPS)
    shift = beta - mean * scale
    return scale.reshape(-1, 1), shift.reshape(-1, 1)


def _dh_stacked_taps(w):
    g, cin = w.shape[0], w.shape[1]
    return jnp.transpose(w, (2, 0, 3, 1)).reshape(3 * g, 3 * cin)


def _pool_matrix(H, W):
    HW, HWp = H * W, (H // 2) * (W // 2)
    r = jnp.arange(HW)
    rp = (r // W // 2) * (W // 2) + (r % W) // 2
    return (rp[:, None] == jnp.arange(HWp)[None, :]).astype(jnp.float32) * 0.25


def kernel(x, conv1_w, dense1_0_bn1_gamma, dense1_0_bn1_beta, dense1_0_bn1_mean, dense1_0_bn1_var, dense1_0_bn2_gamma, dense1_0_bn2_beta, dense1_0_bn2_mean, dense1_0_bn2_var, dense1_0_conv1_w, dense1_0_conv2_w, dense2_0_bn1_gamma, dense2_0_bn1_beta, dense2_0_bn1_mean, dense2_0_bn1_var, dense2_0_bn2_gamma, dense2_0_bn2_beta, dense2_0_bn2_mean, dense2_0_bn2_var, dense2_0_conv1_w, dense2_0_conv2_w, dense3_0_bn1_gamma, dense3_0_bn1_beta, dense3_0_bn1_mean, dense3_0_bn1_var, dense3_0_bn2_gamma, dense3_0_bn2_beta, dense3_0_bn2_mean, dense3_0_bn2_var, dense3_0_conv1_w, dense3_0_conv2_w, trans1_bn_gamma, trans1_bn_beta, trans1_bn_mean, trans1_bn_var, trans1_conv_w, trans2_bn_gamma, trans2_bn_beta, trans2_bn_mean, trans2_bn_var, trans2_conv_w, bn1_gamma, bn1_beta, bn1_mean, bn1_var, fc_w, fc_b):
    N = x.shape[0]
    G = N // _B
    HW1 = _H1 * _H1
    n1 = _B * HW1

    bf = jnp.bfloat16
    xg = (x.reshape(N, 3, HW1).astype(bf)
           .reshape(G, _B, 3, HW1).transpose(0, 2, 1, 3).reshape(G, 3, n1))

    w27 = jnp.transpose(conv1_w, (0, 2, 3, 1)).reshape(conv1_w.shape[0], 27).astype(bf)

    def _dense_prep(bn1g, bn1b, bn1m, bn1v, bn2g, bn2b, bn2m, bn2v, w1, w2):
        s1, b1 = _bn_scale_shift(bn1g, bn1b, bn1m, bn1v)
        s1, b1 = s1.astype(bf), b1.astype(bf)
        s2, b2 = _bn_scale_shift(bn2g, bn2b, bn2m, bn2v)
        w1f = jnp.concatenate([s2 * w1[:, :, 0, 0], b2], axis=1).astype(bf)
        wA = _dh_stacked_taps(w2).astype(bf)
        return s1, b1, w1f, wA

    d1s1, d1b1, d1w1, d1wA = _dense_prep(
        dense1_0_bn1_gamma, dense1_0_bn1_beta, dense1_0_bn1_mean,
        dense1_0_bn1_var, dense1_0_bn2_gamma, dense1_0_bn2_beta,
        dense1_0_bn2_mean, dense1_0_bn2_var, dense1_0_conv1_w, dense1_0_conv2_w)
    d2s1, d2b1, d2w1, d2wA = _dense_prep(
        dense2_0_bn1_gamma, dense2_0_bn1_beta, dense2_0_bn1_mean,
        dense2_0_bn1_var, dense2_0_bn2_gamma, dense2_0_bn2_beta,
        dense2_0_bn2_mean, dense2_0_bn2_var, dense2_0_conv1_w, dense2_0_conv2_w)
    d3s1, d3b1, d3w1, d3wA = _dense_prep(
        dense3_0_bn1_gamma, dense3_0_bn1_beta, dense3_0_bn1_mean,
        dense3_0_bn1_var, dense3_0_bn2_gamma, dense3_0_bn2_beta,
        dense3_0_bn2_mean, dense3_0_bn2_var, dense3_0_conv1_w, dense3_0_conv2_w)

    t1s, t1b = _bn_scale_shift(trans1_bn_gamma, trans1_bn_beta,
                               trans1_bn_mean, trans1_bn_var)
    t1s, t1b = t1s.astype(bf), t1b.astype(bf)
    t1w = trans1_conv_w[:, :, 0, 0].astype(bf)
    p1 = _pool_matrix(_H1, _H1).astype(bf)
    t2s, t2b = _bn_scale_shift(trans2_bn_gamma, trans2_bn_beta,
                               trans2_bn_mean, trans2_bn_var)
    t2s, t2b = t2s.astype(bf), t2b.astype(bf)
    t2w = trans2_conv_w[:, :, 0, 0].astype(bf)
    p2 = _pool_matrix(_H1 // 2, _H1 // 2).astype(bf)

    hs, hb = _bn_scale_shift(bn1_gamma, bn1_beta, bn1_mean, bn1_var)
    nC, Ch = fc_w.shape
    HW3 = (_H1 // 4) * (_H1 // 4)
    mean_m = (jnp.repeat(jnp.eye(_B, dtype=jnp.float32), HW3, axis=0) / HW3)

    full = lambda shp: pl.BlockSpec(shp, lambda g: (0,) * len(shp))
    out = pl.pallas_call(
        _net_kernel,
        out_shape=jax.ShapeDtypeStruct((G, nC, _B), jnp.float32),
        grid=(G,),
        in_specs=[
            pl.BlockSpec((1, 3, n1), lambda g: (g, 0, 0)),
            full(w27.shape),
            full(d1s1.shape), full(d1b1.shape), full(d1w1.shape), full(d1wA.shape),
            full(t1s.shape), full(t1b.shape), full(t1w.shape), full(p1.shape),
            full(d2s1.shape), full(d2b1.shape), full(d2w1.shape), full(d2wA.shape),
            full(t2s.shape), full(t2b.shape), full(t2w.shape), full(p2.shape),
            full(d3s1.shape), full(d3b1.shape), full(d3w1.shape), full(d3wA.shape),
            full(hs.shape), full(hb.shape), full(fc_w.shape),
            full((nC, 1)), full(mean_m.shape),
        ],
        out_specs=pl.BlockSpec((1, nC, _B), lambda g: (g, 0, 0)),
        scratch_shapes=[pltpu.VMEM((384, n1), jnp.bfloat16)],
        compiler_params=pltpu.CompilerParams(dimension_semantics=("parallel",)),
    )(xg, w27,
      d1s1, d1b1, d1w1, d1wA,
      t1s, t1b, t1w, p1,
      d2s1, d2b1, d2w1, d2wA,
      t2s, t2b, t2w, p2,
      d3s1, d3b1, d3w1, d3wA,
      hs, hb, fc_w, fc_b.reshape(nC, 1), mean_m)

    return out.transpose(0, 2, 1).reshape(N, nC)
